# Initial kernel scaffold; baseline (speedup 1.0000x reference)
import jax
import jax.numpy as jnp
from jax.experimental import pallas as pl


def kernel(input_ids, table, W1, b1, gamma, beta, W2, b2):
    raise NotImplementedError("write your pallas kernel here")


# speedup vs baseline: 22.6813x; 22.6813x over previous
import functools
import jax
import jax.numpy as jnp
from jax import lax
from jax.experimental import pallas as pl
from jax.experimental.pallas import tpu as pltpu
from jax.experimental.pallas import tpu_sc as plsc

EMB = 3
H1 = 4
OUT = 5
ROW = 8

NC = 2
NS = 16
NW = NC * NS

COLS = 4096
IDXW = 128
BATCH = 50


def _mlp_table_block(tabT_ref, w1_ref, b1_ref, g_ref, be_ref, w2_ref, b2_ref,
                     out_ref):
    t = [tabT_ref[k:k + 1, :] for k in range(EMB)]
    h = []
    for j in range(H1):
        hj = t[0] * w1_ref[0, j]
        for k in range(1, EMB):
            hj = hj + t[k] * w1_ref[k, j]
        h.append(hj + b1_ref[j])
    mean = (h[0] + h[1] + h[2] + h[3]) * 0.25
    d = [hj - mean for hj in h]
    var = (d[0] * d[0] + d[1] * d[1] + d[2] * d[2] + d[3] * d[3]) * 0.25
    inv = lax.rsqrt(var + 1e-5)
    ln = [d[j] * inv * g_ref[j] + be_ref[j] for j in range(H1)]
    o = []
    for i in range(OUT):
        oi = ln[0] * w2_ref[0, i]
        for j in range(1, H1):
            oi = oi + ln[j] * w2_ref[j, i]
        o.append(oi + b2_ref[i])
    for i in range(OUT, ROW):
        o.append(jnp.zeros_like(o[0]))
    out_ref[...] = jnp.concatenate(o, axis=0).T


def _precompute_table(tabT_pad, W1, b1, gamma, beta, W2, b2, vpad):
    grid = vpad // COLS
    smem = pl.BlockSpec(memory_space=pltpu.SMEM)
    return pl.pallas_call(
        _mlp_table_block,
        grid=(grid,),
        in_specs=[
            pl.BlockSpec((EMB, COLS), lambda i: (0, i)),
            smem, smem, smem, smem, smem, smem,
        ],
        out_specs=pl.BlockSpec((COLS, ROW), lambda i: (i, 0)),
        out_shape=jax.ShapeDtypeStruct((vpad, ROW), jnp.float32),
    )(tabT_pad, W1, b1, gamma, beta, W2, b2)


def _make_gather(n):
    b_per_w = n // NW
    rows_per_w = b_per_w // IDXW
    n_batches = rows_per_w // BATCH
    assert b_per_w * NW == n and rows_per_w * IDXW == b_per_w
    assert n_batches * BATCH == rows_per_w
    bchunk = BATCH * IDXW
    mesh = plsc.VectorSubcoreMesh(core_axis_name="c", subcore_axis_name="s")

    @functools.partial(
        pl.kernel,
        mesh=mesh,
        out_type=jax.ShapeDtypeStruct((n, ROW), jnp.float32),
        scratch_types=[
            pltpu.VMEM((rows_per_w, IDXW), jnp.int32),
            pltpu.VMEM((bchunk, ROW), jnp.float32),
            pltpu.SemaphoreType.DMA,
        ],
        compiler_params=pltpu.CompilerParams(use_tc_tiling_on_sc=False),
    )
    def gather(tab_hbm, idx_hbm, out_hbm, idx_v, rows_v, sem):
        wid = lax.axis_index("s") * NC + lax.axis_index("c")
        base = wid * b_per_w
        pltpu.sync_copy(idx_hbm.at[pl.ds(wid * rows_per_w, rows_per_w)], idx_v)

        def batch_body(g, carry):
            def fire(j, c):
                pltpu.make_async_copy(
                    tab_hbm.at[idx_v.at[g * BATCH + j]],
                    rows_v.at[pl.ds(j * IDXW, IDXW)],
                    sem,
                ).start()
                return c

            lax.fori_loop(0, BATCH, fire, 0)
            pltpu.make_async_copy(
                tab_hbm.at[pl.ds(0, bchunk)], rows_v, sem).wait()
            pltpu.sync_copy(rows_v,
                            out_hbm.at[pl.ds(base + g * bchunk, bchunk)])
            return carry

        lax.fori_loop(0, n_batches, batch_body, 0)

    return gather


def kernel(input_ids, table, W1, b1, gamma, beta, W2, b2):
    b, l = input_ids.shape
    vocab = table.shape[0]
    n = b * l
    ids = input_ids.astype(jnp.int32).reshape(n // IDXW, IDXW)

    vpad = ((vocab + COLS - 1) // COLS) * COLS
    tabT = jnp.pad(table.T, ((0, 0), (0, vpad - vocab)))
    tab_out = _precompute_table(tabT, W1, b1, gamma, beta, W2, b2, vpad)

    flat = _make_gather(n)(tab_out, ids)
    return flat[:, :OUT].reshape(b, l, OUT)
